# Initial kernel scaffold; baseline (speedup 1.0000x reference)
import jax
import jax.numpy as jnp
from jax import lax
from jax.experimental import pallas as pl
from jax.experimental.pallas import tpu as pltpu
from jax.experimental.pallas import tpu_sc as plsc

N = 10000
E = 160000
D = 256
H = 128
NP = 10240
NC = 2
NS = 16
NW = NC * NS

EPW = E // NW
DEG_FULL = EPW // 16
DEG_TAIL = EPW - DEG_FULL * 16

EPT = E // NS
C = 80
NCHUNK = EPT // C
RPT = NP // NS

_mesh = plsc.VectorSubcoreMesh(core_axis_name="c", subcore_axis_name="s")


def _deg_body(dst_hbm, degp_hbm, dstv, hist):
    c = lax.axis_index("c")
    s = lax.axis_index("s")
    wid = s * NC + c

    zero16 = jnp.zeros((16,), jnp.float32)

    @pl.loop(0, NP // 16)
    def _(i):
        hist[pl.ds(i * 16, 16)] = zero16

    base = pl.multiple_of(wid * EPW, 8)
    pltpu.sync_copy(dst_hbm.at[pl.ds(base, EPW)], dstv)

    ones16 = jnp.ones((16,), jnp.float32)

    @pl.loop(0, DEG_FULL)
    def _(j):
        idx = dstv[pl.ds(j * 16, 16)]
        plsc.addupdate_scatter(hist, (idx,), ones16)

    tail = dstv[pl.ds(DEG_FULL * 16 - (16 - DEG_TAIL), 16)]
    mask = lax.iota(jnp.int32, 16) >= (16 - DEG_TAIL)
    plsc.addupdate_scatter(hist, (tail,), ones16, mask=mask)

    pltpu.sync_copy(hist, degp_hbm.at[wid])


def _deg_kernel(dst):
    return pl.kernel(
        _deg_body,
        out_type=jax.ShapeDtypeStruct((NW, NP), jnp.float32),
        mesh=_mesh,
        scratch_types=[
            pltpu.VMEM((EPW,), jnp.int32),
            pltpu.VMEM((NP,), jnp.float32),
        ],
    )(dst)


def _agg_body(src_hbm, dst_hbm, mlo_hbm, mhi_hbm, zrows_hbm,
              out_lo, out_hi, srcv, dstv, rows, agg_sh, sem):
    c = lax.axis_index("c")
    s = lax.axis_index("s")

    pltpu.sync_copy(zrows_hbm, agg_sh.at[pl.ds(s * RPT, RPT)])
    plsc.subcore_barrier()

    base = s * EPT

    @pl.loop(0, NCHUNK)
    def _(j):
        off = pl.multiple_of(base + j * C, 8)
        pltpu.sync_copy(src_hbm.at[pl.ds(off, C)], srcv)
        pltpu.sync_copy(dst_hbm.at[pl.ds(off, C)], dstv)

        @pl.when(c == 0)
        def _():
            pltpu.async_copy(mlo_hbm.at[srcv], rows, sem).wait()

        @pl.when(c == 1)
        def _():
            pltpu.async_copy(mhi_hbm.at[srcv], rows, sem).wait()

        pltpu.sync_copy(rows, agg_sh.at[dstv], add=True)

    plsc.subcore_barrier()

    slc = pl.ds(s * RPT, RPT)

    @pl.when(c == 0)
    def _():
        pltpu.sync_copy(agg_sh.at[slc], out_lo.at[slc])

    @pl.when(c == 1)
    def _():
        pltpu.sync_copy(agg_sh.at[slc], out_hi.at[slc])


def _agg_kernel(src, dst, mlo, mhi, zrows):
    out = jax.ShapeDtypeStruct((NP, H), jnp.float32)
    return pl.kernel(
        _agg_body,
        out_type=(out, out),
        mesh=_mesh,
        scratch_types=[
            pltpu.VMEM((C,), jnp.int32),
            pltpu.VMEM((C,), jnp.int32),
            pltpu.VMEM((C, H), jnp.float32),
            pltpu.VMEM_SHARED((NP, H), jnp.float32),
            pltpu.SemaphoreType.DMA,
        ],
    )(src, dst, mlo, mhi, zrows)


R = 1024
GRID = NP // R


def _dinv(degp_blk):
    deg = jnp.sum(degp_blk, axis=1, keepdims=True)
    return lax.rsqrt(jnp.maximum(deg, 1.0))


def _mm1_body(x_ref, degp_ref, w_ref, lo_ref, hi_ref):
    dinv = _dinv(degp_ref[...])
    m = jnp.dot(x_ref[...], w_ref[...],
                preferred_element_type=jnp.float32) * dinv
    lo_ref[...] = m[:, :H]
    hi_ref[...] = m[:, H:]


def _mm2_body(lo_ref, hi_ref, degp_ref, b_ref, w_ref, olo_ref, ohi_ref):
    dinv = _dinv(degp_ref[...])
    h = jnp.concatenate([lo_ref[...], hi_ref[...]], axis=1) * dinv + b_ref[...]
    m = jnp.dot(h, w_ref[...], preferred_element_type=jnp.float32) * dinv
    olo_ref[...] = m[:, :H]
    ohi_ref[...] = m[:, H:]


def _fin_body(lo_ref, hi_ref, degp_ref, b_ref, o_ref):
    dinv = _dinv(degp_ref[...])
    o_ref[...] = (jnp.concatenate([lo_ref[...], hi_ref[...]], axis=1) * dinv
                  + b_ref[...])


def _row_spec(w):
    return pl.BlockSpec((R, w), lambda i: (i, 0))


def _rep_spec(shp):
    return pl.BlockSpec(shp, lambda i: (0,) * len(shp))


_half_out = (jax.ShapeDtypeStruct((NP, H), jnp.float32),
             jax.ShapeDtypeStruct((NP, H), jnp.float32))


def _mm1(x, degp, w1):
    return pl.pallas_call(
        _mm1_body,
        grid=(GRID,),
        in_specs=[_row_spec(D), _row_spec(NW), _rep_spec((D, D))],
        out_specs=(_row_spec(H), _row_spec(H)),
        out_shape=_half_out,
    )(x, degp, w1)


def _mm2(lo, hi, degp, b1, w2):
    return pl.pallas_call(
        _mm2_body,
        grid=(GRID,),
        in_specs=[_row_spec(H), _row_spec(H), _row_spec(NW),
                  _rep_spec((1, D)), _rep_spec((D, D))],
        out_specs=(_row_spec(H), _row_spec(H)),
        out_shape=_half_out,
    )(lo, hi, degp, b1, w2)


def _fin(lo, hi, degp, b2):
    return pl.pallas_call(
        _fin_body,
        grid=(GRID,),
        in_specs=[_row_spec(H), _row_spec(H), _row_spec(NW),
                  _rep_spec((1, D))],
        out_specs=_row_spec(D),
        out_shape=jax.ShapeDtypeStruct((NP, D), jnp.float32),
    )(lo, hi, degp, b2)


@jax.jit
def kernel(x, edge_index, W1, b1, W2, b2):
    src = edge_index[0]
    dst = edge_index[1]

    x_p = jnp.pad(x, ((0, NP - N), (0, 0)))
    zrows = jnp.zeros((RPT, H), jnp.float32)

    degp = _deg_kernel(dst)
    degp_t = degp.T

    m1_lo, m1_hi = _mm1(x_p, degp_t, W1)
    a1_lo, a1_hi = _agg_kernel(src, dst, m1_lo, m1_hi, zrows)
    m2_lo, m2_hi = _mm2(a1_lo, a1_hi, degp_t, b1.reshape(1, D), W2)
    a2_lo, a2_hi = _agg_kernel(src, dst, m2_lo, m2_hi, zrows)
    out = _fin(a2_lo, a2_hi, degp_t, b2.reshape(1, D))
    return out[:N]


# speedup vs baseline: 6.4469x; 6.4469x over previous
import jax
import jax.numpy as jnp
from jax import lax
from jax.experimental import pallas as pl
from jax.experimental.pallas import tpu as pltpu
from jax.experimental.pallas import tpu_sc as plsc

N = 10000
E = 160000
D = 256
H = 128
NP = 10240
NC = 2
NS = 16
NW = NC * NS

EPW = E // NW
DEG_FULL = EPW // 16
DEG_TAIL = EPW - DEG_FULL * 16

EPT = E // NS
C = 80
NCHUNK = EPT // C
RPT = NP // NS

import functools


@functools.cache
def _mesh():
    return plsc.VectorSubcoreMesh(core_axis_name="c", subcore_axis_name="s",
                                  num_cores=NC, num_subcores=NS)


def _deg_body(dst_hbm, degp_hbm, dstv, hist):
    c = lax.axis_index("c")
    s = lax.axis_index("s")
    wid = s * NC + c

    zero16 = jnp.zeros((16,), jnp.float32)

    @pl.loop(0, NP // 16)
    def _(i):
        hist[pl.ds(i * 16, 16)] = zero16

    dstv[pl.ds(DEG_FULL * 16, 16)] = jnp.zeros((16,), jnp.int32)

    base = pl.multiple_of(wid * EPW, 8)
    pltpu.sync_copy(dst_hbm.at[pl.ds(base, EPW)], dstv.at[pl.ds(0, EPW)])

    ones16 = jnp.ones((16,), jnp.float32)

    @pl.loop(0, DEG_FULL)
    def _(j):
        idx = dstv[pl.ds(j * 16, 16)]
        plsc.addupdate_scatter(hist, (idx,), ones16)

    tail = dstv[pl.ds(DEG_FULL * 16, 16)]
    mask = lax.iota(jnp.int32, 16) < DEG_TAIL
    plsc.addupdate_scatter(hist, (tail,), ones16, mask=mask)

    pltpu.sync_copy(hist, degp_hbm.at[wid])


def _deg_kernel(dst):
    return pl.kernel(
        _deg_body,
        out_type=jax.ShapeDtypeStruct((NW, NP), jnp.float32),
        mesh=_mesh(),
        compiler_params=pltpu.CompilerParams(needs_layout_passes=False),
        scratch_types=[
            pltpu.VMEM((EPW + 16,), jnp.int32),
            pltpu.VMEM((NP,), jnp.float32),
        ],
    )(dst)


def _agg_body(src_hbm, dst_hbm, mlo_hbm, mhi_hbm, zrows_hbm,
              out_lo, out_hi, srcv, dstv, rows, agg_sh, sem):
    c = lax.axis_index("c")
    s = lax.axis_index("s")

    pltpu.sync_copy(zrows_hbm, agg_sh.at[pl.ds(s * RPT, RPT)])
    plsc.subcore_barrier()

    base = s * EPT

    @pl.loop(0, NCHUNK)
    def _(j):
        off = pl.multiple_of(base + j * C, 8)
        pltpu.sync_copy(src_hbm.at[pl.ds(off, C)], srcv)
        pltpu.sync_copy(dst_hbm.at[pl.ds(off, C)], dstv)

        @pl.when(c == 0)
        def _():
            pltpu.async_copy(mlo_hbm.at[srcv], rows, sem).wait()

        @pl.when(c == 1)
        def _():
            pltpu.async_copy(mhi_hbm.at[srcv], rows, sem).wait()

        pltpu.sync_copy(rows, agg_sh.at[dstv], add=True)

    plsc.subcore_barrier()

    slc = pl.ds(s * RPT, RPT)

    @pl.when(c == 0)
    def _():
        pltpu.sync_copy(agg_sh.at[slc], out_lo.at[slc])

    @pl.when(c == 1)
    def _():
        pltpu.sync_copy(agg_sh.at[slc], out_hi.at[slc])


def _agg_kernel(src, dst, mlo, mhi, zrows):
    out = jax.ShapeDtypeStruct((NP, H), jnp.float32)
    return pl.kernel(
        _agg_body,
        out_type=(out, out),
        mesh=_mesh(),
        scratch_types=[
            pltpu.VMEM((C,), jnp.int32),
            pltpu.VMEM((C,), jnp.int32),
            pltpu.VMEM((C, H), jnp.float32),
            pltpu.VMEM_SHARED((NP, H), jnp.float32),
            pltpu.SemaphoreType.DMA,
        ],
    )(src, dst, mlo, mhi, zrows)


R = 1024
GRID = NP // R


def _dinv(degp_blk):
    deg = jnp.sum(degp_blk, axis=1, keepdims=True)
    return lax.rsqrt(jnp.maximum(deg, 1.0))


def _mm1_body(x_ref, degp_ref, w_ref, lo_ref, hi_ref):
    dinv = _dinv(degp_ref[...])
    m = jnp.dot(x_ref[...], w_ref[...],
                preferred_element_type=jnp.float32) * dinv
    lo_ref[...] = m[:, :H]
    hi_ref[...] = m[:, H:]


def _mm2_body(lo_ref, hi_ref, degp_ref, b_ref, w_ref, olo_ref, ohi_ref):
    dinv = _dinv(degp_ref[...])
    h = jnp.concatenate([lo_ref[...], hi_ref[...]], axis=1) * dinv + b_ref[...]
    m = jnp.dot(h, w_ref[...], preferred_element_type=jnp.float32) * dinv
    olo_ref[...] = m[:, :H]
    ohi_ref[...] = m[:, H:]


def _fin_body(lo_ref, hi_ref, degp_ref, b_ref, o_ref):
    dinv = _dinv(degp_ref[...])
    o_ref[...] = (jnp.concatenate([lo_ref[...], hi_ref[...]], axis=1) * dinv
                  + b_ref[...])


def _row_spec(w):
    return pl.BlockSpec((R, w), lambda i: (i, 0))


def _rep_spec(shp):
    return pl.BlockSpec(shp, lambda i: (0,) * len(shp))


_half_out = (jax.ShapeDtypeStruct((NP, H), jnp.float32),
             jax.ShapeDtypeStruct((NP, H), jnp.float32))


def _mm1(x, degp, w1):
    return pl.pallas_call(
        _mm1_body,
        grid=(GRID,),
        in_specs=[_row_spec(D), _row_spec(NW), _rep_spec((D, D))],
        out_specs=(_row_spec(H), _row_spec(H)),
        out_shape=_half_out,
    )(x, degp, w1)


def _mm2(lo, hi, degp, b1, w2):
    return pl.pallas_call(
        _mm2_body,
        grid=(GRID,),
        in_specs=[_row_spec(H), _row_spec(H), _row_spec(NW),
                  _rep_spec((1, D)), _rep_spec((D, D))],
        out_specs=(_row_spec(H), _row_spec(H)),
        out_shape=_half_out,
    )(lo, hi, degp, b1, w2)


def _fin(lo, hi, degp, b2):
    return pl.pallas_call(
        _fin_body,
        grid=(GRID,),
        in_specs=[_row_spec(H), _row_spec(H), _row_spec(NW),
                  _rep_spec((1, D))],
        out_specs=_row_spec(D),
        out_shape=jax.ShapeDtypeStruct((NP, D), jnp.float32),
    )(lo, hi, degp, b2)


@jax.jit
def kernel(x, edge_index, W1, b1, W2, b2):
    src = edge_index[0]
    dst = edge_index[1]

    x_p = jnp.pad(x, ((0, NP - N), (0, 0)))
    zrows = jnp.zeros((RPT, H), jnp.float32)

    degp = _deg_kernel(dst)
    degp_t = degp.T

    m1_lo, m1_hi = _mm1(x_p, degp_t, W1)
    a1_lo, a1_hi = _agg_kernel(src, dst, m1_lo, m1_hi, zrows)
    m2_lo, m2_hi = _mm2(a1_lo, a1_hi, degp_t, b1.reshape(1, D), W2)
    a2_lo, a2_hi = _agg_kernel(src, dst, m2_lo, m2_hi, zrows)
    out = _fin(a2_lo, a2_hi, degp_t, b2.reshape(1, D))
    return out[:N]


# speedup vs baseline: 13.8928x; 2.1550x over previous
import jax
import jax.numpy as jnp
from jax import lax
from jax.experimental import pallas as pl
from jax.experimental.pallas import tpu as pltpu
from jax.experimental.pallas import tpu_sc as plsc

N = 10000
E = 160000
D = 256
H = 128
NP = 10240
NC = 2
NS = 16
NW = NC * NS

EPW = E // NW
DEG_FULL = EPW // 16
DEG_TAIL = EPW - DEG_FULL * 16

EPT = E // NS
C = 80
NCHUNK = EPT // C
RPT = NP // NS

import functools


@functools.cache
def _mesh():
    return plsc.VectorSubcoreMesh(core_axis_name="c", subcore_axis_name="s",
                                  num_cores=NC, num_subcores=NS)


def _deg_body(dst_hbm, degp_hbm, dstv, hist):
    c = lax.axis_index("c")
    s = lax.axis_index("s")
    wid = s * NC + c

    zero16 = jnp.zeros((16,), jnp.float32)

    @pl.loop(0, NP // 16)
    def _(i):
        hist[pl.ds(i * 16, 16)] = zero16

    dstv[pl.ds(DEG_FULL * 16, 16)] = jnp.zeros((16,), jnp.int32)

    base = pl.multiple_of(wid * EPW, 8)
    pltpu.sync_copy(dst_hbm.at[pl.ds(base, EPW)], dstv.at[pl.ds(0, EPW)])

    ones16 = jnp.ones((16,), jnp.float32)

    @pl.loop(0, DEG_FULL)
    def _(j):
        idx = dstv[pl.ds(j * 16, 16)]
        plsc.addupdate_scatter(hist, (idx,), ones16)

    tail = dstv[pl.ds(DEG_FULL * 16, 16)]
    mask = lax.iota(jnp.int32, 16) < DEG_TAIL
    plsc.addupdate_scatter(hist, (tail,), ones16, mask=mask)

    pltpu.sync_copy(hist, degp_hbm.at[wid])


def _deg_kernel(dst):
    return pl.kernel(
        _deg_body,
        out_type=jax.ShapeDtypeStruct((NW, NP), jnp.float32),
        mesh=_mesh(),
        compiler_params=pltpu.CompilerParams(needs_layout_passes=False),
        scratch_types=[
            pltpu.VMEM((EPW + 16,), jnp.int32),
            pltpu.VMEM((NP,), jnp.float32),
        ],
    )(dst)


def _agg_body(src3_hbm, dst3_hbm, mlo_hbm, mhi_hbm, zrows_hbm,
              out_lo, out_hi, srcv, dstv, rows0, rows1, agg_sh, sem0, sem1):
    c = lax.axis_index("c")
    s = lax.axis_index("s")

    pltpu.sync_copy(src3_hbm.at[pl.ds(s * EPT, EPT)], srcv)
    pltpu.sync_copy(dst3_hbm.at[s], dstv)
    pltpu.sync_copy(zrows_hbm, agg_sh.at[pl.ds(s * RPT, RPT)])
    plsc.subcore_barrier()

    def gather_start(j, buf, sem):
        idx = srcv.at[pl.ds(pl.multiple_of(j * C, 8), C)]

        @pl.when(c == 0)
        def _():
            pltpu.async_copy(mlo_hbm.at[idx], buf, sem)

        @pl.when(c == 1)
        def _():
            pltpu.async_copy(mhi_hbm.at[idx], buf, sem)

    def gather_wait(buf, sem):
        pltpu.make_async_copy(mlo_hbm.at[pl.ds(0, C)], buf, sem).wait()

    def scatter(j, buf):
        pltpu.sync_copy(buf, agg_sh.at[dstv.at[j]], add=True)

    gather_start(0, rows0, sem0)

    @pl.loop(0, (NCHUNK - 1) // 2)
    def _(j2):
        j = 2 * j2
        gather_start(j + 1, rows1, sem1)
        gather_wait(rows0, sem0)
        scatter(j, rows0)

        @pl.when(j + 2 < NCHUNK)
        def _():
            gather_start(j + 2, rows0, sem0)

        gather_wait(rows1, sem1)
        scatter(j + 1, rows1)

    if NCHUNK % 2 == 1:
        gather_wait(rows0, sem0)
        scatter(NCHUNK - 1, rows0)

    plsc.subcore_barrier()

    slc = pl.ds(s * RPT, RPT)

    @pl.when(c == 0)
    def _():
        pltpu.sync_copy(agg_sh.at[slc], out_lo.at[slc])

    @pl.when(c == 1)
    def _():
        pltpu.sync_copy(agg_sh.at[slc], out_hi.at[slc])


def _agg_kernel(src3, dst3, mlo, mhi, zrows):
    out = jax.ShapeDtypeStruct((NP, H), jnp.float32)
    return pl.kernel(
        _agg_body,
        out_type=(out, out),
        mesh=_mesh(),
        compiler_params=pltpu.CompilerParams(needs_layout_passes=False),
        scratch_types=[
            pltpu.VMEM((EPT,), jnp.int32),
            pltpu.VMEM((NCHUNK, C), jnp.int32),
            pltpu.VMEM((C, H), jnp.float32),
            pltpu.VMEM((C, H), jnp.float32),
            pltpu.VMEM_SHARED((NP, H), jnp.float32),
            pltpu.SemaphoreType.DMA,
            pltpu.SemaphoreType.DMA,
        ],
    )(src3, dst3, mlo, mhi, zrows)


R = 1024
GRID = NP // R


def _dinv(degp_blk):
    deg = jnp.sum(degp_blk, axis=1, keepdims=True)
    return lax.rsqrt(jnp.maximum(deg, 1.0))


def _mm1_body(x_ref, degp_ref, w_ref, lo_ref, hi_ref):
    dinv = _dinv(degp_ref[...])
    m = jnp.dot(x_ref[...], w_ref[...],
                preferred_element_type=jnp.float32) * dinv
    lo_ref[...] = m[:, :H]
    hi_ref[...] = m[:, H:]


def _mm2_body(lo_ref, hi_ref, degp_ref, b_ref, w_ref, olo_ref, ohi_ref):
    dinv = _dinv(degp_ref[...])
    h = jnp.concatenate([lo_ref[...], hi_ref[...]], axis=1) * dinv + b_ref[...]
    m = jnp.dot(h, w_ref[...], preferred_element_type=jnp.float32) * dinv
    olo_ref[...] = m[:, :H]
    ohi_ref[...] = m[:, H:]


def _fin_body(lo_ref, hi_ref, degp_ref, b_ref, o_ref):
    dinv = _dinv(degp_ref[...])
    o_ref[...] = (jnp.concatenate([lo_ref[...], hi_ref[...]], axis=1) * dinv
                  + b_ref[...])


def _row_spec(w):
    return pl.BlockSpec((R, w), lambda i: (i, 0))


def _rep_spec(shp):
    return pl.BlockSpec(shp, lambda i: (0,) * len(shp))


_half_out = (jax.ShapeDtypeStruct((NP, H), jnp.float32),
             jax.ShapeDtypeStruct((NP, H), jnp.float32))


def _mm1(x, degp, w1):
    return pl.pallas_call(
        _mm1_body,
        grid=(GRID,),
        in_specs=[_row_spec(D), _row_spec(NW), _rep_spec((D, D))],
        out_specs=(_row_spec(H), _row_spec(H)),
        out_shape=_half_out,
    )(x, degp, w1)


def _mm2(lo, hi, degp, b1, w2):
    return pl.pallas_call(
        _mm2_body,
        grid=(GRID,),
        in_specs=[_row_spec(H), _row_spec(H), _row_spec(NW),
                  _rep_spec((1, D)), _rep_spec((D, D))],
        out_specs=(_row_spec(H), _row_spec(H)),
        out_shape=_half_out,
    )(lo, hi, degp, b1, w2)


def _fin(lo, hi, degp, b2):
    return pl.pallas_call(
        _fin_body,
        grid=(GRID,),
        in_specs=[_row_spec(H), _row_spec(H), _row_spec(NW),
                  _rep_spec((1, D))],
        out_specs=_row_spec(D),
        out_shape=jax.ShapeDtypeStruct((NP, D), jnp.float32),
    )(lo, hi, degp, b2)


@jax.jit
def kernel(x, edge_index, W1, b1, W2, b2):
    src = edge_index[0]
    dst = edge_index[1]
    src3 = src
    dst3 = dst.reshape(NS, NCHUNK, C)

    x_p = jnp.pad(x, ((0, NP - N), (0, 0)))
    zrows = jnp.zeros((RPT, H), jnp.float32)

    degp = _deg_kernel(dst)
    degp_t = degp.T

    m1_lo, m1_hi = _mm1(x_p, degp_t, W1)
    a1_lo, a1_hi = _agg_kernel(src3, dst3, m1_lo, m1_hi, zrows)
    m2_lo, m2_hi = _mm2(a1_lo, a1_hi, degp_t, b1.reshape(1, D), W2)
    a2_lo, a2_hi = _agg_kernel(src3, dst3, m2_lo, m2_hi, zrows)
    out = _fin(a2_lo, a2_hi, degp_t, b2.reshape(1, D))
    return out[:N]
